# Initial kernel scaffold; baseline (speedup 1.0000x reference)
import jax
import jax.numpy as jnp
from jax.experimental import pallas as pl


def kernel(xyz, feats, W1, gn1_w, gn1_b, W2, gn2_w, gn2_b, Wp, gnp_w, gnp_b):
    raise NotImplementedError("write your pallas kernel here")


# speedup vs baseline: 6.2377x; 6.2377x over previous
import functools

import jax
import jax.numpy as jnp
from jax import lax
from jax.experimental import pallas as pl
from jax.experimental.pallas import tpu as pltpu
from jax.experimental.pallas import tpu_sc as plsc

B = 4
N = 16384
IN_C = 64
OUT_C = 128
K = 16
S = 2048
MID = 64
DT = 128
TQ = 256
ROWS = 8
COLS = N // ROWS


def _fps_kernel(xyz_ref, sxyz_ref):
    x0 = xyz_ref[0, 0:ROWS, :]
    x1 = xyz_ref[0, ROWS:2 * ROWS, :]
    x2 = xyz_ref[0, 2 * ROWS:3 * ROWS, :]
    r_iota = lax.broadcasted_iota(jnp.int32, (ROWS, COLS), 0)
    c_iota = lax.broadcasted_iota(jnp.int32, (ROWS, COLS), 1)
    idx2d = r_iota * COLS + c_iota

    sxyz_ref[0, 3:8, :] = jnp.zeros((5, S), jnp.float32)
    s_iota = lax.broadcasted_iota(jnp.int32, (1, S), 1)

    def body(i, carry):
        dists, far, c0s, c1s, c2s = carry
        sel = idx2d == far
        zero = jnp.zeros((ROWS, COLS), jnp.float32)
        c0 = jnp.sum(jnp.where(sel, x0, zero), keepdims=True).reshape(1, 1)
        c1 = jnp.sum(jnp.where(sel, x1, zero), keepdims=True).reshape(1, 1)
        c2 = jnp.sum(jnp.where(sel, x2, zero), keepdims=True).reshape(1, 1)
        d0 = x0 - c0
        d1 = x1 - c1
        d2 = x2 - c2
        d = d0 * d0 + d1 * d1 + d2 * d2
        dists = jnp.minimum(dists, d)
        m = jnp.max(dists, keepdims=True).reshape(1, 1)
        far_new = jnp.min(
            jnp.where(dists == m, idx2d, jnp.int32(N)), keepdims=True
        ).reshape(1, 1)
        here = s_iota == i
        c0s = jnp.where(here, c0, c0s)
        c1s = jnp.where(here, c1, c1s)
        c2s = jnp.where(here, c2, c2s)
        return dists, far_new, c0s, c1s, c2s

    d0 = jnp.full((ROWS, COLS), 1e10, jnp.float32)
    zrow = jnp.zeros((1, S), jnp.float32)
    _, _, c0s, c1s, c2s = lax.fori_loop(
        0, S, body, (d0, jnp.zeros((1, 1), jnp.int32), zrow, zrow, zrow)
    )
    sxyz_ref[0, 0:1, :] = c0s
    sxyz_ref[0, 1:2, :] = c1s
    sxyz_ref[0, 2:3, :] = c2s


def _run_fps(xyz_fold):
    return pl.pallas_call(
        _fps_kernel,
        grid=(B,),
        in_specs=[pl.BlockSpec((1, 3 * ROWS, COLS), lambda b: (b, 0, 0))],
        out_specs=pl.BlockSpec((1, 8, S), lambda b: (b, 0, 0)),
        out_shape=jax.ShapeDtypeStruct((B, 8, S), jnp.float32),
    )(xyz_fold)


def _knn_kernel(sxyz_ref, xyzp_ref, idx_ref):
    b = pl.program_id(0)
    q8 = jnp.transpose(sxyz_ref[0], (1, 0))
    x8 = xyzp_ref[0]
    qn = jnp.sum(q8 * q8, axis=1, keepdims=True)
    xn = jnp.sum(x8 * x8, axis=0, keepdims=True)
    m = jnp.dot(q8, x8, preferred_element_type=jnp.float32)
    d = (qn - 2.0 * m) + xn
    l_iota = lax.broadcasted_iota(jnp.int32, (TQ, N), 1)
    k_iota = lax.broadcasted_iota(jnp.int32, (TQ, 128), 1)
    off = (b * N).astype(jnp.int32)

    def body(k, carry):
        dc, acc = carry
        mn = jnp.min(dc, axis=1, keepdims=True)
        amin = jnp.min(
            jnp.where(dc == mn, l_iota, jnp.int32(N)), axis=1, keepdims=True
        )
        acc = jnp.where(k_iota == k, amin + off, acc)
        dc = jnp.where(l_iota == amin, jnp.float32(1e30), dc)
        return dc, acc

    _, acc = lax.fori_loop(
        0, K, body, (d, jnp.zeros((TQ, 128), jnp.int32))
    )
    idx_ref[0] = acc


def _run_knn(sxyz, xyz_pad):
    return pl.pallas_call(
        _knn_kernel,
        grid=(B, S // TQ),
        in_specs=[
            pl.BlockSpec((1, 8, TQ), lambda b, t: (b, 0, t)),
            pl.BlockSpec((1, 8, N), lambda b, t: (b, 0, 0)),
        ],
        out_specs=pl.BlockSpec((1, TQ, 128), lambda b, t: (b, t, 0)),
        out_shape=jax.ShapeDtypeStruct((B, S, 128), jnp.int32),
    )(sxyz, xyz_pad)


def _gather_sc(table, idx_flat):
    info = plsc.get_sparse_core_info()
    nw = info.num_cores * info.num_subcores
    tot = idx_flat.shape[0]
    b_per_w = tot // nw
    chunk = 512
    n_chunks = b_per_w // chunk
    mesh = plsc.VectorSubcoreMesh(core_axis_name="c", subcore_axis_name="s")

    @functools.partial(
        pl.kernel,
        mesh=mesh,
        out_type=jax.ShapeDtypeStruct((tot, DT), jnp.float32),
        scratch_types=[
            pltpu.VMEM((chunk,), jnp.int32),
            pltpu.VMEM((chunk, DT), jnp.float32),
            pltpu.SemaphoreType.DMA,
        ],
    )
    def gk(table_hbm, idx_hbm, out_hbm, idx_v, rows_v, sem):
        wid = lax.axis_index("s") * info.num_cores + lax.axis_index("c")
        base = wid * b_per_w
        for j in range(n_chunks):
            off = base + j * chunk
            pltpu.sync_copy(idx_hbm.at[pl.ds(off, chunk)], idx_v)
            pltpu.async_copy(table_hbm.at[idx_v], rows_v, sem).wait()
            pltpu.sync_copy(rows_v, out_hbm.at[pl.ds(off, chunk)])

    return gk(table, idx_flat)


CH = 1024
NCH = (S * K) // CH
QC = CH // K


def _mlp_kernel(g_ref, s_ref, w1_ref, w2_ref, wp_ref, m1_ref, m2_ref,
                g1w_ref, g1b_ref, g2w_ref, g2b_ref, gpw_ref, gpb_ref,
                out_ref, x2s, ys):
    w1 = w1_ref[:]
    w2 = w2_ref[:]
    m1 = m1_ref[:]
    m2 = m2_ref[:]
    eps = jnp.float32(1e-5)

    def xin_chunk(c):
        xg = g_ref[0, pl.ds(c * CH, CH), :]
        sg = s_ref[0, pl.ds(c * QC, QC), :]
        xg3 = xg.reshape(QC, K, DT)
        xin = xg3 - sg.reshape(QC, 1, DT)
        return xin.reshape(CH, DT)

    def p1(c, carry):
        cs, cs2 = carry
        x1 = jnp.dot(xin_chunk(c), w1, preferred_element_type=jnp.float32)
        cs = cs + jnp.sum(x1, axis=0, keepdims=True)
        cs2 = cs2 + jnp.sum(x1 * x1, axis=0, keepdims=True)
        return cs, cs2

    z64 = jnp.zeros((1, MID), jnp.float32)
    cs, cs2 = lax.fori_loop(0, NCH, p1, (z64, z64))
    inv1 = jnp.float32(1.0 / (S * K * 8))
    mean1 = jnp.dot(cs, m1, preferred_element_type=jnp.float32) * inv1
    e21 = jnp.dot(cs2, m1, preferred_element_type=jnp.float32) * inv1
    var1 = e21 - mean1 * mean1
    sc1 = g1w_ref[:] * lax.rsqrt(var1 + eps)
    bi1 = g1b_ref[:] - mean1 * sc1

    def p2(c, carry):
        ds, ds2 = carry
        x1 = jnp.dot(xin_chunk(c), w1, preferred_element_type=jnp.float32)
        x1n = jnp.maximum(x1 * sc1 + bi1, 0.0)
        x2 = jnp.dot(x1n, w2, preferred_element_type=jnp.float32)
        ds = ds + jnp.sum(x2, axis=0, keepdims=True)
        ds2 = ds2 + jnp.sum(x2 * x2, axis=0, keepdims=True)
        x2s[pl.ds(c * CH, CH), :] = x2
        return ds, ds2

    z128 = jnp.zeros((1, OUT_C), jnp.float32)
    ds, ds2 = lax.fori_loop(0, NCH, p2, (z128, z128))
    inv2 = jnp.float32(1.0 / (S * K * 16))
    mean2 = jnp.dot(ds, m2, preferred_element_type=jnp.float32) * inv2
    e22 = jnp.dot(ds2, m2, preferred_element_type=jnp.float32) * inv2
    var2 = e22 - mean2 * mean2
    sc2 = g2w_ref[:] * lax.rsqrt(var2 + eps)
    bi2 = g2b_ref[:] - mean2 * sc2

    def p3(c, _):
        x2 = x2s[pl.ds(c * CH, CH), :]
        x2n = jnp.maximum(x2 * sc2 + bi2, 0.0)
        y = jnp.max(x2n.reshape(QC, K, OUT_C), axis=1)
        ys[pl.ds(c * QC, QC), :] = y
        return 0

    lax.fori_loop(0, NCH, p3, 0)

    z = jnp.dot(ys[:], wp_ref[:], preferred_element_type=jnp.float32)
    es = jnp.sum(z, axis=0, keepdims=True)
    es2 = jnp.sum(z * z, axis=0, keepdims=True)
    invp = jnp.float32(1.0 / (S * 16))
    meanp = jnp.dot(es, m2, preferred_element_type=jnp.float32) * invp
    e2p = jnp.dot(es2, m2, preferred_element_type=jnp.float32) * invp
    varp = e2p - meanp * meanp
    scp = gpw_ref[:] * lax.rsqrt(varp + eps)
    bip = gpb_ref[:] - meanp * scp
    out_ref[0] = jnp.maximum(z * scp + bip, 0.0)


def _run_mlp(g, s_pad, w1p, w2, wp, m1, m2, g1w, g1b, g2w, g2b, gpw, gpb):
    vec = lambda n: pl.BlockSpec((1, n), lambda b: (0, 0))
    mat = lambda r, c: pl.BlockSpec((r, c), lambda b: (0, 0))
    return pl.pallas_call(
        _mlp_kernel,
        grid=(B,),
        in_specs=[
            pl.BlockSpec((1, S * K, DT), lambda b: (b, 0, 0)),
            pl.BlockSpec((1, S, DT), lambda b: (b, 0, 0)),
            mat(DT, MID), mat(MID, OUT_C), mat(OUT_C, OUT_C),
            mat(MID, MID), mat(OUT_C, OUT_C),
            vec(MID), vec(MID), vec(OUT_C), vec(OUT_C), vec(OUT_C),
            vec(OUT_C),
        ],
        out_specs=pl.BlockSpec((1, S, OUT_C), lambda b: (b, 0, 0)),
        out_shape=jax.ShapeDtypeStruct((B, S, OUT_C), jnp.float32),
        scratch_shapes=[
            pltpu.VMEM((S * K, OUT_C), jnp.float32),
            pltpu.VMEM((S, OUT_C), jnp.float32),
        ],
    )(g, s_pad, w1p, w2, wp, m1, m2, g1w, g1b, g2w, g2b, gpw, gpb)


@jax.jit
def kernel(xyz, feats, W1, gn1_w, gn1_b, W2, gn2_w, gn2_b, Wp, gnp_w, gnp_b):
    xyz_t = jnp.transpose(xyz, (0, 2, 1))
    xyz_fold = xyz_t.reshape(B, 3 * ROWS, COLS)
    sxyz8 = _run_fps(xyz_fold)

    xyz_pad = jnp.concatenate(
        [xyz_t, jnp.zeros((B, 5, N), jnp.float32)], axis=1
    )
    idx128 = _run_knn(sxyz8, xyz_pad)
    idx_flat = idx128[:, :, :K].reshape(B * S * K)

    table = jnp.concatenate(
        [feats, xyz, jnp.zeros((B, N, DT - IN_C - 3), jnp.float32)], axis=-1
    ).reshape(B * N, DT)
    g = _gather_sc(table, idx_flat).reshape(B, S * K, DT)

    s_xyz = jnp.transpose(sxyz8, (0, 2, 1))[:, :, :3]
    s_pad = jnp.concatenate(
        [
            jnp.zeros((B, S, IN_C), jnp.float32),
            s_xyz,
            jnp.zeros((B, S, DT - IN_C - 3), jnp.float32),
        ],
        axis=-1,
    )

    w1p = jnp.concatenate(
        [W1, jnp.zeros((DT - IN_C - 3, MID), jnp.float32)], axis=0
    )
    gidx1 = jnp.arange(MID, dtype=jnp.int32) // 8
    m1 = (gidx1[:, None] == gidx1[None, :]).astype(jnp.float32)
    gidx2 = jnp.arange(OUT_C, dtype=jnp.int32) // 16
    m2 = (gidx2[:, None] == gidx2[None, :]).astype(jnp.float32)

    out = _run_mlp(
        g, s_pad, w1p, W2, Wp, m1, m2,
        gn1_w.reshape(1, MID), gn1_b.reshape(1, MID),
        gn2_w.reshape(1, OUT_C), gn2_b.reshape(1, OUT_C),
        gnp_w.reshape(1, OUT_C), gnp_b.reshape(1, OUT_C),
    )
    return (s_xyz, out)
